# baseline (speedup 1.0000x reference)
import functools

import jax
import jax.numpy as jnp
from jax.experimental import pallas as pl
from jax.experimental.pallas import tpu as pltpu


def _fused_kernel(x_ref, w_ref, adj_ref, b_ref, o_ref, wq_ref, sup_ref, *,
                  wgt_alpha, act_alpha, w_levels, a_levels, n_elem, nx, ck):
    j = pl.program_id(1)

    @pl.when(j == 0)
    def _():
        w = w_ref[...]
        mean = jnp.sum(w) / n_elem
        var = jnp.sum((w - mean) ** 2) / (n_elem - 1.0)
        w_n = (w - mean) / jnp.sqrt(var)
        wc = jnp.clip(w_n / wgt_alpha, -1.0, 1.0)
        w_q = (jnp.round(jnp.abs(wc) * w_levels) / w_levels) \
            * jnp.sign(wc) * wgt_alpha
        wq_ref[...] = w_q.astype(jnp.bfloat16)

    @pl.when(j < nx)
    def _():
        xc = jnp.minimum(x_ref[...] / act_alpha, 1.0)
        x_q = (jnp.round(xc * a_levels) / a_levels) * act_alpha
        sup_ref[pl.ds(j * ck, ck), :] = jnp.dot(
            x_q.astype(jnp.bfloat16), wq_ref[...],
            preferred_element_type=jnp.float32).astype(jnp.bfloat16)

    @pl.when(j >= nx)
    def _():
        o_ref[...] = jnp.dot(
            adj_ref[...].astype(jnp.bfloat16), sup_ref[...],
            preferred_element_type=jnp.float32) + b_ref[...]


def kernel(x, adj, weight, bias):
    f32 = jnp.float32
    x = x.astype(f32)
    adj = adj.astype(f32)
    weight = weight.astype(f32)
    n, fin = x.shape
    fout = weight.shape[1]
    b2 = bias.astype(f32).reshape(1, fout)

    cores = 2
    tile = min(1024, n // cores)
    ntiles = (n // cores) // tile
    ck = min(2048, n)
    nx = n // ck
    assert cores * ntiles * tile == n and nx * ck == n

    out = pl.pallas_call(
        functools.partial(
            _fused_kernel, wgt_alpha=3.0, act_alpha=1.0,
            w_levels=7.0, a_levels=15.0, n_elem=float(fin * fout),
            nx=nx, ck=ck),
        out_shape=jax.ShapeDtypeStruct((n, fout), f32),
        grid=(cores, nx + ntiles),
        in_specs=[
            pl.BlockSpec((ck, fin),
                         lambda i, j: (jnp.minimum(j, nx - 1), 0)),
            pl.BlockSpec((fin, fout), lambda i, j: (0, 0)),
            pl.BlockSpec((tile, n),
                         lambda i, j: (i * ntiles + jnp.clip(j - nx, 0, ntiles - 1), 0)),
            pl.BlockSpec((1, fout), lambda i, j: (0, 0)),
        ],
        out_specs=pl.BlockSpec(
            (tile, fout),
            lambda i, j: (i * ntiles + jnp.clip(j - nx, 0, ntiles - 1), 0)),
        scratch_shapes=[
            pltpu.VMEM((fin, fout), jnp.bfloat16),
            pltpu.VMEM((n, fout), jnp.bfloat16),
        ],
        compiler_params=pltpu.CompilerParams(
            dimension_semantics=("parallel", "arbitrary"),
            vmem_limit_bytes=48 * 1024 * 1024),
        cost_estimate=pl.CostEstimate(
            flops=2 * n * n * fout + 2 * n * fin * fout,
            transcendentals=0,
            bytes_accessed=4 * (n * n + n * fin + fin * fout
                                + n * fout + fout)),
    )(x, weight, adj, b2)
    return out
